# Initial kernel scaffold; baseline (speedup 1.0000x reference)
import jax
import jax.numpy as jnp
from jax.experimental import pallas as pl


def kernel(x, edge_index, e, i, Wf0, bf0, Ws0, bs0, Wf1, bf1, Ws1, bs1, Wf2, bf2, Ws2, bs2, W1, b1, W2, b2, W3, b3):
    raise NotImplementedError("write your pallas kernel here")


# speedup vs baseline: 1.9845x; 1.9845x over previous
import functools

import jax
import jax.numpy as jnp
from jax import lax
from jax.experimental import pallas as pl
from jax.experimental.pallas import tpu as pltpu
from jax.experimental.pallas import tpu_sc as plsc

N_NODES = 10000
N_EDGES = 320000
D = 128
D_EDGE = 16
N_GRAPHS = 256

NC = 2
NS = 16
NW = NC * NS
E_PER_W = N_EDGES // NW
K = 80
N_CHUNKS = E_PER_W // K
N_PAD = 10240
NODES_PER_TILE = N_PAD // NS

_sc_mesh = plsc.VectorSubcoreMesh(core_axis_name="c", subcore_axis_name="s")


@functools.partial(
    pl.kernel,
    out_type=[
        jax.ShapeDtypeStruct((N_EDGES, 2 * D), jnp.float32),
        jax.ShapeDtypeStruct((N_EDGES, 2 * D), jnp.float32),
    ],
    mesh=_sc_mesh,
    scratch_types=[
        pltpu.VMEM((K,), jnp.int32),
        pltpu.VMEM((K,), jnp.int32),
        pltpu.VMEM((K, 2 * D), jnp.float32),
        pltpu.VMEM((K, 2 * D), jnp.float32),
        pltpu.SemaphoreType.DMA,
        pltpu.SemaphoreType.DMA,
    ],
)
def _sc_gather(p_hbm, q_hbm, row_hbm, col_hbm, u1_hbm, u2_hbm,
               ridx_v, cidx_v, prow_v, qrow_v, sem1, sem2):
    wid = lax.axis_index("s") * NC + lax.axis_index("c")
    base = wid * E_PER_W

    def body(ci, carry):
        off = base + ci * K
        pltpu.sync_copy(row_hbm.at[pl.ds(off, K)], ridx_v)
        pltpu.sync_copy(col_hbm.at[pl.ds(off, K)], cidx_v)
        d1 = pltpu.async_copy(p_hbm.at[ridx_v], prow_v, sem1)
        d2 = pltpu.async_copy(q_hbm.at[cidx_v], qrow_v, sem2)
        d1.wait()
        d2.wait()
        pltpu.sync_copy(prow_v, u1_hbm.at[pl.ds(off, K)])
        pltpu.sync_copy(qrow_v, u2_hbm.at[pl.ds(off, K)])
        return carry

    lax.fori_loop(0, N_CHUNKS, body, 0)


@functools.partial(
    pl.kernel,
    out_type=jax.ShapeDtypeStruct((NC, N_PAD, D), jnp.float32),
    mesh=_sc_mesh,
    scratch_types=[
        pltpu.VMEM((K,), jnp.int32),
        pltpu.VMEM((K, D), jnp.float32),
        pltpu.VMEM_SHARED((N_PAD, D), jnp.float32),
        pltpu.SemaphoreType.DMA,
    ],
)
def _sc_scatter(msg_hbm, row_hbm, zeros_hbm, out_hbm, idx_v, msg_v, acc_sh, sem):
    c = lax.axis_index("c")
    s = lax.axis_index("s")
    stripe = s * NODES_PER_TILE
    pltpu.sync_copy(zeros_hbm.at[pl.ds(stripe, NODES_PER_TILE)],
                    acc_sh.at[pl.ds(stripe, NODES_PER_TILE)])
    plsc.subcore_barrier()

    base = c * (N_EDGES // NC) + s * E_PER_W

    def body(ci, carry):
        off = base + ci * K
        pltpu.sync_copy(row_hbm.at[pl.ds(off, K)], idx_v)
        pltpu.sync_copy(msg_hbm.at[pl.ds(off, K)], msg_v)
        pltpu.sync_copy(msg_v, acc_sh.at[idx_v], add=True)
        return carry

    lax.fori_loop(0, N_CHUNKS, body, 0)
    plsc.subcore_barrier()
    pltpu.sync_copy(acc_sh.at[pl.ds(stripe, NODES_PER_TILE)],
                    out_hbm.at[c, pl.ds(stripe, NODES_PER_TILE)])


BM = 2000


def _proj0_body(h_ref, wr_ref, wc_ref, p_ref, q_ref):
    h = h_ref[...]
    p_ref[...] = jnp.dot(h, wr_ref[...], preferred_element_type=jnp.float32)
    q_ref[...] = jnp.dot(h, wc_ref[...], preferred_element_type=jnp.float32)


def _proj_body(h_ref, a_ref, wr_ref, wc_ref, hn_ref, p_ref, q_ref):
    h = h_ref[...] + a_ref[0] + a_ref[1]
    hn_ref[...] = h
    p_ref[...] = jnp.dot(h, wr_ref[...], preferred_element_type=jnp.float32)
    q_ref[...] = jnp.dot(h, wc_ref[...], preferred_element_type=jnp.float32)


def _tc_proj0(h, wr, wc):
    return pl.pallas_call(
        _proj0_body,
        grid=(N_NODES // BM,),
        in_specs=[
            pl.BlockSpec((BM, D), lambda m: (m, 0)),
            pl.BlockSpec((D, 2 * D), lambda m: (0, 0)),
            pl.BlockSpec((D, 2 * D), lambda m: (0, 0)),
        ],
        out_specs=[
            pl.BlockSpec((BM, 2 * D), lambda m: (m, 0)),
            pl.BlockSpec((BM, 2 * D), lambda m: (m, 0)),
        ],
        out_shape=[
            jax.ShapeDtypeStruct((N_NODES, 2 * D), jnp.float32),
            jax.ShapeDtypeStruct((N_NODES, 2 * D), jnp.float32),
        ],
    )(h, wr, wc)


def _tc_proj(h, aggs, wr, wc):
    return pl.pallas_call(
        _proj_body,
        grid=(N_NODES // BM,),
        in_specs=[
            pl.BlockSpec((BM, D), lambda m: (m, 0)),
            pl.BlockSpec((NC, BM, D), lambda m: (0, m, 0)),
            pl.BlockSpec((D, 2 * D), lambda m: (0, 0)),
            pl.BlockSpec((D, 2 * D), lambda m: (0, 0)),
        ],
        out_specs=[
            pl.BlockSpec((BM, D), lambda m: (m, 0)),
            pl.BlockSpec((BM, 2 * D), lambda m: (m, 0)),
            pl.BlockSpec((BM, 2 * D), lambda m: (m, 0)),
        ],
        out_shape=[
            jax.ShapeDtypeStruct((N_NODES, D), jnp.float32),
            jax.ShapeDtypeStruct((N_NODES, 2 * D), jnp.float32),
            jax.ShapeDtypeStruct((N_NODES, 2 * D), jnp.float32),
        ],
    )(h, aggs, wr, wc)


BE = 4000


def _edge_body(u1_ref, u2_ref, e_ref, we_ref, b_ref, msg_ref):
    t = (u1_ref[...] + u2_ref[...] + b_ref[...]
         + jnp.dot(e_ref[...], we_ref[...], preferred_element_type=jnp.float32))
    gate = jax.nn.sigmoid(t[:, :D])
    core = jax.nn.softplus(t[:, D:])
    msg_ref[...] = gate * core


def _tc_edge(u1, u2, e, we, bcat):
    return pl.pallas_call(
        _edge_body,
        grid=(N_EDGES // BE,),
        in_specs=[
            pl.BlockSpec((BE, 2 * D), lambda m: (m, 0)),
            pl.BlockSpec((BE, 2 * D), lambda m: (m, 0)),
            pl.BlockSpec((BE, D_EDGE), lambda m: (m, 0)),
            pl.BlockSpec((D_EDGE, 2 * D), lambda m: (0, 0)),
            pl.BlockSpec((1, 2 * D), lambda m: (0, 0)),
        ],
        out_specs=pl.BlockSpec((BE, D), lambda m: (m, 0)),
        out_shape=jax.ShapeDtypeStruct((N_EDGES, D), jnp.float32),
    )(u1, u2, e, we, bcat)


def _head_body(h_ref, a_ref, gid_ref, w1_ref, b1_ref, w2_ref, b2_ref,
               w3_ref, b3_ref, out_ref):
    h4 = h_ref[...] + a_ref[0, :N_NODES] + a_ref[1, :N_NODES]
    seg = lax.broadcasted_iota(jnp.int32, (N_GRAPHS, N_NODES), 0)
    m = (seg == gid_ref[...]).astype(jnp.float32)
    sums = jnp.dot(m, h4, preferred_element_type=jnp.float32)
    counts = jnp.sum(m, axis=1, keepdims=True)
    pooled = sums / jnp.maximum(counts, 1.0)
    o = jnp.maximum(jnp.dot(pooled, w1_ref[...],
                            preferred_element_type=jnp.float32) + b1_ref[...], 0.0)
    o = jnp.maximum(jnp.dot(o, w2_ref[...],
                            preferred_element_type=jnp.float32) + b2_ref[...], 0.0)
    out_ref[...] = jax.nn.sigmoid(
        jnp.dot(o, w3_ref[...], preferred_element_type=jnp.float32) + b3_ref[...])


def _tc_head(h, aggs, gid, w1, b1, w2, b2, w3, b3):
    return pl.pallas_call(
        _head_body,
        out_shape=jax.ShapeDtypeStruct((N_GRAPHS, 1), jnp.float32),
    )(h, aggs, gid, w1, b1, w2, b2, w3, b3)


def kernel(x, edge_index, e, i, Wf0, bf0, Ws0, bs0, Wf1, bf1, Ws1, bs1,
           Wf2, bf2, Ws2, bs2, W1, b1, W2, b2, W3, b3):
    row = edge_index[0]
    col = edge_index[1]
    gid = i.reshape(1, N_NODES)
    zeros_nodes = jnp.zeros((N_PAD, D), jnp.float32)

    layers = []
    for Wf, bf, Ws, bs in ((Wf0, bf0, Ws0, bs0), (Wf1, bf1, Ws1, bs1),
                           (Wf2, bf2, Ws2, bs2)):
        wr = jnp.concatenate([Wf[:D], Ws[:D]], axis=1)
        wc = jnp.concatenate([Wf[D:2 * D], Ws[D:2 * D]], axis=1)
        we = jnp.concatenate([Wf[2 * D:], Ws[2 * D:]], axis=1)
        bcat = jnp.concatenate([bf, bs]).reshape(1, 2 * D)
        layers.append((wr, wc, we, bcat))

    h = x
    aggs = None
    for wr, wc, we, bcat in layers:
        if aggs is None:
            p, q = _tc_proj0(h, wr, wc)
        else:
            h, p, q = _tc_proj(h, aggs, wr, wc)
        u1, u2 = _sc_gather(p, q, row, col)
        msg = _tc_edge(u1, u2, e, we, bcat)
        aggs = _sc_scatter(msg, row, zeros_nodes)

    return _tc_head(h, aggs, gid, W1, b1.reshape(1, D), W2, b2.reshape(1, D),
                    W3, b3.reshape(1, 1))


# speedup vs baseline: 3.9556x; 1.9932x over previous
import functools

import jax
import jax.numpy as jnp
from jax import lax
from jax.experimental import pallas as pl
from jax.experimental.pallas import tpu as pltpu
from jax.experimental.pallas import tpu_sc as plsc

N_NODES = 10000
N_EDGES = 320000
D = 128
D_EDGE = 16
N_GRAPHS = 256

NC = 2
NS = 16
NW = NC * NS
E_PER_W = N_EDGES // NW
K = 80
N_CHUNKS = E_PER_W // K
N_PAD = 10240
NODES_PER_TILE = N_PAD // NS

_sc_mesh = plsc.VectorSubcoreMesh(core_axis_name="c", subcore_axis_name="s")


NBUF = 5
GDIST = 3
SNBUF = 3
SDIST = 2


@functools.partial(
    pl.kernel,
    out_type=[
        jax.ShapeDtypeStruct((N_EDGES, D), jnp.float32),
        jax.ShapeDtypeStruct((N_EDGES, D), jnp.float32),
    ],
    mesh=_sc_mesh,
    scratch_types=(
        [
            pltpu.VMEM((E_PER_W,), jnp.int32),
            pltpu.VMEM((E_PER_W,), jnp.int32),
            pltpu.VMEM((NBUF, K, D), jnp.float32),
            pltpu.VMEM((NBUF, K, D), jnp.float32),
        ]
        + [pltpu.SemaphoreType.DMA] * (2 * NBUF)
    ),
)
def _sc_gather(p_hbm, q_hbm, row_hbm, col_hbm, u1_hbm, u2_hbm,
               ridx_v, cidx_v, pbuf, qbuf, *sems):
    sem_g = sems[:NBUF]
    sem_w = sems[NBUF:]
    wid = lax.axis_index("s") * NC + lax.axis_index("c")
    base = wid * E_PER_W

    pltpu.sync_copy(row_hbm.at[pl.ds(base, E_PER_W)], ridx_v)
    pltpu.sync_copy(col_hbm.at[pl.ds(base, E_PER_W)], cidx_v)

    def start_gather(b, chunk):
        pltpu.async_copy(p_hbm.at[ridx_v.at[pl.ds(chunk * K, K)]],
                         pbuf.at[b], sem_g[b])
        pltpu.async_copy(q_hbm.at[cidx_v.at[pl.ds(chunk * K, K)]],
                         qbuf.at[b], sem_g[b])

    def wait_gather(b):
        pltpu.make_async_copy(p_hbm.at[pl.ds(0, K)], pbuf.at[b], sem_g[b]).wait()
        pltpu.make_async_copy(q_hbm.at[pl.ds(0, K)], qbuf.at[b], sem_g[b]).wait()

    def start_wb(b, chunk):
        off = base + chunk * K
        pltpu.async_copy(pbuf.at[b], u1_hbm.at[pl.ds(off, K)], sem_w[b])
        pltpu.async_copy(qbuf.at[b], u2_hbm.at[pl.ds(off, K)], sem_w[b])

    def wait_wb(b):
        pltpu.make_async_copy(pbuf.at[b], u1_hbm.at[pl.ds(base, K)], sem_w[b]).wait()
        pltpu.make_async_copy(qbuf.at[b], u2_hbm.at[pl.ds(base, K)], sem_w[b]).wait()

    for b in range(GDIST):
        start_gather(b, b)

    def outer(g, carry):
        for b in range(NBUF):
            c = g * NBUF + b
            nxt = c + GDIST
            nb = (b + GDIST) % NBUF

            @pl.when(nxt < N_CHUNKS)
            def _():
                @pl.when(nxt >= NBUF)
                def _():
                    wait_wb(nb)
                start_gather(nb, nxt)

            wait_gather(b)
            start_wb(b, c)
        return carry

    lax.fori_loop(0, N_CHUNKS // NBUF, outer, 0)
    for b in range(NBUF):
        wait_wb(b)


@functools.partial(
    pl.kernel,
    out_type=jax.ShapeDtypeStruct((NC, N_PAD, D), jnp.float32),
    mesh=_sc_mesh,
    scratch_types=(
        [
            pltpu.VMEM((E_PER_W,), jnp.int32),
            pltpu.VMEM((SNBUF, K, D), jnp.float32),
            pltpu.VMEM_SHARED((N_PAD, D), jnp.float32),
        ]
        + [pltpu.SemaphoreType.DMA] * SNBUF
    ),
)
def _sc_scatter(msg_hbm, row_hbm, zeros_hbm, out_hbm, idx_v, mbuf, acc_sh, *sems):
    c = lax.axis_index("c")
    s = lax.axis_index("s")
    stripe = s * NODES_PER_TILE
    pltpu.sync_copy(zeros_hbm.at[pl.ds(stripe, NODES_PER_TILE)],
                    acc_sh.at[pl.ds(stripe, NODES_PER_TILE)])

    base = c * (N_EDGES // NC) + s * E_PER_W
    pltpu.sync_copy(row_hbm.at[pl.ds(base, E_PER_W)], idx_v)
    plsc.subcore_barrier()

    def start_load(b, chunk):
        pltpu.async_copy(msg_hbm.at[pl.ds(base + chunk * K, K)],
                         mbuf.at[b], sems[b])

    def wait_load(b):
        pltpu.make_async_copy(msg_hbm.at[pl.ds(base, K)], mbuf.at[b],
                              sems[b]).wait()

    def consume(b, chunk):
        wait_load(b)
        pltpu.sync_copy(mbuf.at[b],
                        acc_sh.at[idx_v.at[pl.ds(chunk * K, K)]], add=True)

    for b in range(SDIST):
        start_load(b, b)

    n_main = (N_CHUNKS // SNBUF) * SNBUF

    def outer(g, carry):
        for b in range(SNBUF):
            chunk = g * SNBUF + b
            nxt = chunk + SDIST
            nb = (b + SDIST) % SNBUF

            @pl.when(nxt < N_CHUNKS)
            def _():
                start_load(nb, nxt)

            consume(b, chunk)
        return carry

    lax.fori_loop(0, n_main // SNBUF, outer, 0)
    for chunk in range(n_main, N_CHUNKS):
        consume(chunk % SNBUF, chunk)
    plsc.subcore_barrier()
    pltpu.sync_copy(acc_sh.at[pl.ds(stripe, NODES_PER_TILE)],
                    out_hbm.at[c, pl.ds(stripe, NODES_PER_TILE)])


BM = 2000


def _pack_bf16(gate_part, core_part):
    gb = lax.bitcast_convert_type(
        gate_part.astype(jnp.bfloat16).astype(jnp.float32), jnp.uint32)
    cb = lax.bitcast_convert_type(
        core_part.astype(jnp.bfloat16).astype(jnp.float32), jnp.uint32)
    word = (gb >> 16) | (cb & jnp.uint32(0xFFFF0000))
    return lax.bitcast_convert_type(word, jnp.float32)


def _unpack_bf16(packed):
    w = lax.bitcast_convert_type(packed, jnp.uint32)
    gate_part = lax.bitcast_convert_type(w << 16, jnp.float32)
    core_part = lax.bitcast_convert_type(w & jnp.uint32(0xFFFF0000), jnp.float32)
    return gate_part, core_part


def _proj0_body(h_ref, wr_ref, wc_ref, p_ref, q_ref):
    h = h_ref[...]
    tp = jnp.dot(h, wr_ref[...], preferred_element_type=jnp.float32)
    tq = jnp.dot(h, wc_ref[...], preferred_element_type=jnp.float32)
    p_ref[...] = _pack_bf16(tp[:, :D], tp[:, D:])
    q_ref[...] = _pack_bf16(tq[:, :D], tq[:, D:])


def _proj_body(h_ref, a_ref, wr_ref, wc_ref, hn_ref, p_ref, q_ref):
    h = h_ref[...] + a_ref[0] + a_ref[1]
    hn_ref[...] = h
    tp = jnp.dot(h, wr_ref[...], preferred_element_type=jnp.float32)
    tq = jnp.dot(h, wc_ref[...], preferred_element_type=jnp.float32)
    p_ref[...] = _pack_bf16(tp[:, :D], tp[:, D:])
    q_ref[...] = _pack_bf16(tq[:, :D], tq[:, D:])


def _tc_proj0(h, wr, wc):
    return pl.pallas_call(
        _proj0_body,
        grid=(N_NODES // BM,),
        in_specs=[
            pl.BlockSpec((BM, D), lambda m: (m, 0)),
            pl.BlockSpec((D, 2 * D), lambda m: (0, 0)),
            pl.BlockSpec((D, 2 * D), lambda m: (0, 0)),
        ],
        out_specs=[
            pl.BlockSpec((BM, D), lambda m: (m, 0)),
            pl.BlockSpec((BM, D), lambda m: (m, 0)),
        ],
        out_shape=[
            jax.ShapeDtypeStruct((N_NODES, D), jnp.float32),
            jax.ShapeDtypeStruct((N_NODES, D), jnp.float32),
        ],
    )(h, wr, wc)


def _tc_proj(h, aggs, wr, wc):
    return pl.pallas_call(
        _proj_body,
        grid=(N_NODES // BM,),
        in_specs=[
            pl.BlockSpec((BM, D), lambda m: (m, 0)),
            pl.BlockSpec((NC, BM, D), lambda m: (0, m, 0)),
            pl.BlockSpec((D, 2 * D), lambda m: (0, 0)),
            pl.BlockSpec((D, 2 * D), lambda m: (0, 0)),
        ],
        out_specs=[
            pl.BlockSpec((BM, D), lambda m: (m, 0)),
            pl.BlockSpec((BM, D), lambda m: (m, 0)),
            pl.BlockSpec((BM, D), lambda m: (m, 0)),
        ],
        out_shape=[
            jax.ShapeDtypeStruct((N_NODES, D), jnp.float32),
            jax.ShapeDtypeStruct((N_NODES, D), jnp.float32),
            jax.ShapeDtypeStruct((N_NODES, D), jnp.float32),
        ],
    )(h, aggs, wr, wc)


BE = 4000


def _edge_body(u1_ref, u2_ref, e_ref, we_ref, b_ref, msg_ref):
    g1, c1 = _unpack_bf16(u1_ref[...])
    g2, c2 = _unpack_bf16(u2_ref[...])
    ep = (jnp.dot(e_ref[...], we_ref[...], preferred_element_type=jnp.float32)
          + b_ref[...])
    gate = jax.nn.sigmoid(g1 + g2 + ep[:, :D])
    core = jax.nn.softplus(c1 + c2 + ep[:, D:])
    msg_ref[...] = gate * core


def _tc_edge(u1, u2, e, we, bcat):
    return pl.pallas_call(
        _edge_body,
        grid=(N_EDGES // BE,),
        in_specs=[
            pl.BlockSpec((BE, D), lambda m: (m, 0)),
            pl.BlockSpec((BE, D), lambda m: (m, 0)),
            pl.BlockSpec((BE, D_EDGE), lambda m: (m, 0)),
            pl.BlockSpec((D_EDGE, 2 * D), lambda m: (0, 0)),
            pl.BlockSpec((1, 2 * D), lambda m: (0, 0)),
        ],
        out_specs=pl.BlockSpec((BE, D), lambda m: (m, 0)),
        out_shape=jax.ShapeDtypeStruct((N_EDGES, D), jnp.float32),
    )(u1, u2, e, we, bcat)


def _head_body(h_ref, a_ref, gid_ref, w1_ref, b1_ref, w2_ref, b2_ref,
               w3_ref, b3_ref, out_ref):
    h4 = h_ref[...] + a_ref[0, :N_NODES] + a_ref[1, :N_NODES]
    seg = lax.broadcasted_iota(jnp.int32, (N_GRAPHS, N_NODES), 0)
    m = (seg == gid_ref[...]).astype(jnp.float32)
    sums = jnp.dot(m, h4, preferred_element_type=jnp.float32)
    counts = jnp.sum(m, axis=1, keepdims=True)
    pooled = sums / jnp.maximum(counts, 1.0)
    o = jnp.maximum(jnp.dot(pooled, w1_ref[...],
                            preferred_element_type=jnp.float32) + b1_ref[...], 0.0)
    o = jnp.maximum(jnp.dot(o, w2_ref[...],
                            preferred_element_type=jnp.float32) + b2_ref[...], 0.0)
    out_ref[...] = jax.nn.sigmoid(
        jnp.dot(o, w3_ref[...], preferred_element_type=jnp.float32) + b3_ref[...])


def _tc_head(h, aggs, gid, w1, b1, w2, b2, w3, b3):
    return pl.pallas_call(
        _head_body,
        out_shape=jax.ShapeDtypeStruct((N_GRAPHS, 1), jnp.float32),
    )(h, aggs, gid, w1, b1, w2, b2, w3, b3)


def kernel(x, edge_index, e, i, Wf0, bf0, Ws0, bs0, Wf1, bf1, Ws1, bs1,
           Wf2, bf2, Ws2, bs2, W1, b1, W2, b2, W3, b3):
    row = edge_index[0]
    col = edge_index[1]
    gid = i.reshape(1, N_NODES)
    zeros_nodes = jnp.zeros((N_PAD, D), jnp.float32)

    layers = []
    for Wf, bf, Ws, bs in ((Wf0, bf0, Ws0, bs0), (Wf1, bf1, Ws1, bs1),
                           (Wf2, bf2, Ws2, bs2)):
        wr = jnp.concatenate([Wf[:D], Ws[:D]], axis=1)
        wc = jnp.concatenate([Wf[D:2 * D], Ws[D:2 * D]], axis=1)
        we = jnp.concatenate([Wf[2 * D:], Ws[2 * D:]], axis=1)
        bcat = jnp.concatenate([bf, bs]).reshape(1, 2 * D)
        layers.append((wr, wc, we, bcat))

    h = x
    aggs = None
    for wr, wc, we, bcat in layers:
        if aggs is None:
            p, q = _tc_proj0(h, wr, wc)
        else:
            h, p, q = _tc_proj(h, aggs, wr, wc)
        u1, u2 = _sc_gather(p, q, row, col)
        msg = _tc_edge(u1, u2, e, we, bcat)
        aggs = _sc_scatter(msg, row, zeros_nodes)

    return _tc_head(h, aggs, gid, W1, b1.reshape(1, D), W2, b2.reshape(1, D),
                    W3, b3.reshape(1, 1))


# speedup vs baseline: 4.0245x; 1.0174x over previous
import functools

import jax
import jax.numpy as jnp
from jax import lax
from jax.experimental import pallas as pl
from jax.experimental.pallas import tpu as pltpu
from jax.experimental.pallas import tpu_sc as plsc

N_NODES = 10000
N_EDGES = 320000
D = 128
D_EDGE = 16
N_GRAPHS = 256

NC = 2
NS = 16
NW = NC * NS
EH = N_EDGES // 2
E_PER_W = EH // NW
K = 40
N_CHUNKS = E_PER_W // K
N_PAD = 10240
NODES_PER_TILE = N_PAD // NS

_sc_mesh = plsc.VectorSubcoreMesh(core_axis_name="c", subcore_axis_name="s")


NBUF = 5
GDIST = 3
SNBUF = 3
SDIST = 2


@functools.partial(
    pl.kernel,
    out_type=[
        jax.ShapeDtypeStruct((EH, D), jnp.float32),
        jax.ShapeDtypeStruct((EH, D), jnp.float32),
    ],
    mesh=_sc_mesh,
    scratch_types=(
        [
            pltpu.VMEM((E_PER_W,), jnp.int32),
            pltpu.VMEM((E_PER_W,), jnp.int32),
            pltpu.VMEM((NBUF, K, D), jnp.float32),
            pltpu.VMEM((NBUF, K, D), jnp.float32),
        ]
        + [pltpu.SemaphoreType.DMA] * (2 * NBUF)
    ),
)
def _sc_gather(p_hbm, q_hbm, row_hbm, col_hbm, u1_hbm, u2_hbm,
               ridx_v, cidx_v, pbuf, qbuf, *sems):
    sem_g = sems[:NBUF]
    sem_w = sems[NBUF:]
    wid = lax.axis_index("s") * NC + lax.axis_index("c")
    base = wid * E_PER_W

    pltpu.sync_copy(row_hbm.at[pl.ds(base, E_PER_W)], ridx_v)
    pltpu.sync_copy(col_hbm.at[pl.ds(base, E_PER_W)], cidx_v)

    def start_gather(b, chunk):
        pltpu.async_copy(p_hbm.at[ridx_v.at[pl.ds(chunk * K, K)]],
                         pbuf.at[b], sem_g[b])
        pltpu.async_copy(q_hbm.at[cidx_v.at[pl.ds(chunk * K, K)]],
                         qbuf.at[b], sem_g[b])

    def wait_gather(b):
        pltpu.make_async_copy(p_hbm.at[pl.ds(0, K)], pbuf.at[b], sem_g[b]).wait()
        pltpu.make_async_copy(q_hbm.at[pl.ds(0, K)], qbuf.at[b], sem_g[b]).wait()

    def start_wb(b, chunk):
        off = base + chunk * K
        pltpu.async_copy(pbuf.at[b], u1_hbm.at[pl.ds(off, K)], sem_w[b])
        pltpu.async_copy(qbuf.at[b], u2_hbm.at[pl.ds(off, K)], sem_w[b])

    def wait_wb(b):
        pltpu.make_async_copy(pbuf.at[b], u1_hbm.at[pl.ds(base, K)], sem_w[b]).wait()
        pltpu.make_async_copy(qbuf.at[b], u2_hbm.at[pl.ds(base, K)], sem_w[b]).wait()

    for b in range(GDIST):
        start_gather(b, b)

    def outer(g, carry):
        for b in range(NBUF):
            c = g * NBUF + b
            nxt = c + GDIST
            nb = (b + GDIST) % NBUF

            @pl.when(nxt < N_CHUNKS)
            def _():
                @pl.when(nxt >= NBUF)
                def _():
                    wait_wb(nb)
                start_gather(nb, nxt)

            wait_gather(b)
            start_wb(b, c)
        return carry

    lax.fori_loop(0, N_CHUNKS // NBUF, outer, 0)
    for b in range(NBUF):
        wait_wb(b)


@functools.partial(
    pl.kernel,
    out_type=jax.ShapeDtypeStruct((NC, N_PAD, D), jnp.float32),
    mesh=_sc_mesh,
    scratch_types=(
        [
            pltpu.VMEM((E_PER_W,), jnp.int32),
            pltpu.VMEM((SNBUF, K, D), jnp.float32),
            pltpu.VMEM_SHARED((N_PAD, D), jnp.float32),
        ]
        + [pltpu.SemaphoreType.DMA] * SNBUF
    ),
)
def _sc_scatter(msg_hbm, row_hbm, init_hbm, out_hbm, idx_v, mbuf, acc_sh, *sems):
    c = lax.axis_index("c")
    s = lax.axis_index("s")
    stripe = s * NODES_PER_TILE
    pltpu.sync_copy(init_hbm.at[c, pl.ds(stripe, NODES_PER_TILE)],
                    acc_sh.at[pl.ds(stripe, NODES_PER_TILE)])

    base = c * (EH // NC) + s * E_PER_W
    pltpu.sync_copy(row_hbm.at[pl.ds(base, E_PER_W)], idx_v)
    plsc.subcore_barrier()

    def start_load(b, chunk):
        pltpu.async_copy(msg_hbm.at[pl.ds(base + chunk * K, K)],
                         mbuf.at[b], sems[b])

    def wait_load(b):
        pltpu.make_async_copy(msg_hbm.at[pl.ds(base, K)], mbuf.at[b],
                              sems[b]).wait()

    def consume(b, chunk):
        wait_load(b)
        pltpu.sync_copy(mbuf.at[b],
                        acc_sh.at[idx_v.at[pl.ds(chunk * K, K)]], add=True)

    for b in range(SDIST):
        start_load(b, b)

    n_main = (N_CHUNKS // SNBUF) * SNBUF

    def outer(g, carry):
        for b in range(SNBUF):
            chunk = g * SNBUF + b
            nxt = chunk + SDIST
            nb = (b + SDIST) % SNBUF

            @pl.when(nxt < N_CHUNKS)
            def _():
                start_load(nb, nxt)

            consume(b, chunk)
        return carry

    lax.fori_loop(0, n_main // SNBUF, outer, 0)
    for chunk in range(n_main, N_CHUNKS):
        consume(chunk % SNBUF, chunk)
    plsc.subcore_barrier()
    pltpu.sync_copy(acc_sh.at[pl.ds(stripe, NODES_PER_TILE)],
                    out_hbm.at[c, pl.ds(stripe, NODES_PER_TILE)])


BM = 2000


def _pack_bf16(gate_part, core_part):
    gb = lax.bitcast_convert_type(
        gate_part.astype(jnp.bfloat16).astype(jnp.float32), jnp.uint32)
    cb = lax.bitcast_convert_type(
        core_part.astype(jnp.bfloat16).astype(jnp.float32), jnp.uint32)
    word = (gb >> 16) | (cb & jnp.uint32(0xFFFF0000))
    return lax.bitcast_convert_type(word, jnp.float32)


def _unpack_bf16(packed):
    w = lax.bitcast_convert_type(packed, jnp.uint32)
    gate_part = lax.bitcast_convert_type(w << 16, jnp.float32)
    core_part = lax.bitcast_convert_type(w & jnp.uint32(0xFFFF0000), jnp.float32)
    return gate_part, core_part


def _proj0_body(h_ref, wr_ref, wc_ref, p_ref, q_ref):
    h = h_ref[...]
    tp = jnp.dot(h, wr_ref[...], preferred_element_type=jnp.float32)
    tq = jnp.dot(h, wc_ref[...], preferred_element_type=jnp.float32)
    p_ref[...] = _pack_bf16(tp[:, :D], tp[:, D:])
    q_ref[...] = _pack_bf16(tq[:, :D], tq[:, D:])


def _proj_body(h_ref, a_ref, wr_ref, wc_ref, hn_ref, p_ref, q_ref):
    h = h_ref[...] + a_ref[0] + a_ref[1]
    hn_ref[...] = h
    tp = jnp.dot(h, wr_ref[...], preferred_element_type=jnp.float32)
    tq = jnp.dot(h, wc_ref[...], preferred_element_type=jnp.float32)
    p_ref[...] = _pack_bf16(tp[:, :D], tp[:, D:])
    q_ref[...] = _pack_bf16(tq[:, :D], tq[:, D:])


def _tc_proj0(h, wr, wc):
    return pl.pallas_call(
        _proj0_body,
        grid=(N_NODES // BM,),
        in_specs=[
            pl.BlockSpec((BM, D), lambda m: (m, 0)),
            pl.BlockSpec((D, 2 * D), lambda m: (0, 0)),
            pl.BlockSpec((D, 2 * D), lambda m: (0, 0)),
        ],
        out_specs=[
            pl.BlockSpec((BM, D), lambda m: (m, 0)),
            pl.BlockSpec((BM, D), lambda m: (m, 0)),
        ],
        out_shape=[
            jax.ShapeDtypeStruct((N_NODES, D), jnp.float32),
            jax.ShapeDtypeStruct((N_NODES, D), jnp.float32),
        ],
    )(h, wr, wc)


def _tc_proj(h, aggs, wr, wc):
    return pl.pallas_call(
        _proj_body,
        grid=(N_NODES // BM,),
        in_specs=[
            pl.BlockSpec((BM, D), lambda m: (m, 0)),
            pl.BlockSpec((NC, BM, D), lambda m: (0, m, 0)),
            pl.BlockSpec((D, 2 * D), lambda m: (0, 0)),
            pl.BlockSpec((D, 2 * D), lambda m: (0, 0)),
        ],
        out_specs=[
            pl.BlockSpec((BM, D), lambda m: (m, 0)),
            pl.BlockSpec((BM, D), lambda m: (m, 0)),
            pl.BlockSpec((BM, D), lambda m: (m, 0)),
        ],
        out_shape=[
            jax.ShapeDtypeStruct((N_NODES, D), jnp.float32),
            jax.ShapeDtypeStruct((N_NODES, D), jnp.float32),
            jax.ShapeDtypeStruct((N_NODES, D), jnp.float32),
        ],
    )(h, aggs, wr, wc)


BE = 4000


def _edge_body(u1_ref, u2_ref, e_ref, we_ref, b_ref, msg_ref):
    g1, c1 = _unpack_bf16(u1_ref[...])
    g2, c2 = _unpack_bf16(u2_ref[...])
    ep = (jnp.dot(e_ref[...], we_ref[...], preferred_element_type=jnp.float32)
          + b_ref[...])
    gate = jax.nn.sigmoid(g1 + g2 + ep[:, :D])
    core = jax.nn.softplus(c1 + c2 + ep[:, D:])
    msg_ref[...] = gate * core


def _tc_edge(u1, u2, e, we, bcat):
    return pl.pallas_call(
        _edge_body,
        grid=(EH // BE,),
        in_specs=[
            pl.BlockSpec((BE, D), lambda m: (m, 0)),
            pl.BlockSpec((BE, D), lambda m: (m, 0)),
            pl.BlockSpec((BE, D_EDGE), lambda m: (m, 0)),
            pl.BlockSpec((D_EDGE, 2 * D), lambda m: (0, 0)),
            pl.BlockSpec((1, 2 * D), lambda m: (0, 0)),
        ],
        out_specs=pl.BlockSpec((BE, D), lambda m: (m, 0)),
        out_shape=jax.ShapeDtypeStruct((EH, D), jnp.float32),
    )(u1, u2, e, we, bcat)


def _head_body(h_ref, a_ref, gid_ref, w1_ref, b1_ref, w2_ref, b2_ref,
               w3_ref, b3_ref, out_ref):
    h4 = h_ref[...] + a_ref[0, :N_NODES] + a_ref[1, :N_NODES]
    seg = lax.broadcasted_iota(jnp.int32, (N_GRAPHS, N_NODES), 0)
    m = (seg == gid_ref[...]).astype(jnp.float32)
    sums = jnp.dot(m, h4, preferred_element_type=jnp.float32)
    counts = jnp.sum(m, axis=1, keepdims=True)
    pooled = sums / jnp.maximum(counts, 1.0)
    o = jnp.maximum(jnp.dot(pooled, w1_ref[...],
                            preferred_element_type=jnp.float32) + b1_ref[...], 0.0)
    o = jnp.maximum(jnp.dot(o, w2_ref[...],
                            preferred_element_type=jnp.float32) + b2_ref[...], 0.0)
    out_ref[...] = jax.nn.sigmoid(
        jnp.dot(o, w3_ref[...], preferred_element_type=jnp.float32) + b3_ref[...])


def _tc_head(h, aggs, gid, w1, b1, w2, b2, w3, b3):
    return pl.pallas_call(
        _head_body,
        out_shape=jax.ShapeDtypeStruct((N_GRAPHS, 1), jnp.float32),
    )(h, aggs, gid, w1, b1, w2, b2, w3, b3)


def kernel(x, edge_index, e, i, Wf0, bf0, Ws0, bs0, Wf1, bf1, Ws1, bs1,
           Wf2, bf2, Ws2, bs2, W1, b1, W2, b2, W3, b3):
    row = edge_index[0]
    col = edge_index[1]
    row_h = (row[:EH], row[EH:])
    col_h = (col[:EH], col[EH:])
    e_h = (e[:EH], e[EH:])
    gid = i.reshape(1, N_NODES)
    zeros_parts = jnp.zeros((NC, N_PAD, D), jnp.float32)

    layers = []
    for Wf, bf, Ws, bs in ((Wf0, bf0, Ws0, bs0), (Wf1, bf1, Ws1, bs1),
                           (Wf2, bf2, Ws2, bs2)):
        wr = jnp.concatenate([Wf[:D], Ws[:D]], axis=1)
        wc = jnp.concatenate([Wf[D:2 * D], Ws[D:2 * D]], axis=1)
        we = jnp.concatenate([Wf[2 * D:], Ws[2 * D:]], axis=1)
        bcat = jnp.concatenate([bf, bs]).reshape(1, 2 * D)
        layers.append((wr, wc, we, bcat))

    h = x
    aggs = None
    for wr, wc, we, bcat in layers:
        if aggs is None:
            p, q = _tc_proj0(h, wr, wc)
        else:
            h, p, q = _tc_proj(h, aggs, wr, wc)
        u1a, u2a = _sc_gather(p, q, row_h[0], col_h[0])
        u1b, u2b = _sc_gather(p, q, row_h[1], col_h[1])
        msg_a = _tc_edge(u1a, u2a, e_h[0], we, bcat)
        msg_b = _tc_edge(u1b, u2b, e_h[1], we, bcat)
        part_a = _sc_scatter(msg_a, row_h[0], zeros_parts)
        aggs = _sc_scatter(msg_b, row_h[1], part_a)

    return _tc_head(h, aggs, gid, W1, b1.reshape(1, D), W2, b2.reshape(1, D),
                    W3, b3.reshape(1, 1))


# speedup vs baseline: 4.0341x; 1.0024x over previous
import functools

import jax
import jax.numpy as jnp
from jax import lax
from jax.experimental import pallas as pl
from jax.experimental.pallas import tpu as pltpu
from jax.experimental.pallas import tpu_sc as plsc

N_NODES = 10000
N_EDGES = 320000
D = 128
D_EDGE = 16
N_GRAPHS = 256

NC = 2
NS = 16
NW = NC * NS
EH = N_EDGES // 2
E_PER_W = EH // NW
K = 128
NCH = E_PER_W // K
TAIL = E_PER_W - NCH * K
TAIL_OFF = NCH * K
N_PAD = 10240
NODES_PER_TILE = N_PAD // NS

_sc_mesh = plsc.VectorSubcoreMesh(core_axis_name="c", subcore_axis_name="s")


NBUF = 3
GDIST = 2
SNBUF = 2
SDIST = 1


@functools.partial(
    pl.kernel,
    out_type=[
        jax.ShapeDtypeStruct((EH, D), jnp.float32),
        jax.ShapeDtypeStruct((EH, D), jnp.float32),
    ],
    mesh=_sc_mesh,
    scratch_types=(
        [
            pltpu.VMEM((E_PER_W,), jnp.int32),
            pltpu.VMEM((E_PER_W,), jnp.int32),
            pltpu.VMEM((NBUF, K, D), jnp.float32),
            pltpu.VMEM((NBUF, K, D), jnp.float32),
        ]
        + [pltpu.SemaphoreType.DMA] * (2 * NBUF)
    ),
)
def _sc_gather(p_hbm, q_hbm, row_hbm, col_hbm, u1_hbm, u2_hbm,
               ridx_v, cidx_v, pbuf, qbuf, *sems):
    sem_g = sems[:NBUF]
    sem_w = sems[NBUF:]
    wid = lax.axis_index("s") * NC + lax.axis_index("c")
    base = wid * E_PER_W

    pltpu.sync_copy(row_hbm.at[pl.ds(base, E_PER_W)], ridx_v)
    pltpu.sync_copy(col_hbm.at[pl.ds(base, E_PER_W)], cidx_v)

    def start_gather(b, chunk, n=K):
        pltpu.async_copy(p_hbm.at[ridx_v.at[pl.ds(chunk * K, n)]],
                         pbuf.at[b, pl.ds(0, n)], sem_g[b])
        pltpu.async_copy(q_hbm.at[cidx_v.at[pl.ds(chunk * K, n)]],
                         qbuf.at[b, pl.ds(0, n)], sem_g[b])

    def wait_gather(b, n=K):
        pltpu.make_async_copy(p_hbm.at[pl.ds(0, n)],
                              pbuf.at[b, pl.ds(0, n)], sem_g[b]).wait()
        pltpu.make_async_copy(q_hbm.at[pl.ds(0, n)],
                              qbuf.at[b, pl.ds(0, n)], sem_g[b]).wait()

    def start_wb(b, chunk, n=K):
        off = base + chunk * K
        pltpu.async_copy(pbuf.at[b, pl.ds(0, n)],
                         u1_hbm.at[pl.ds(off, n)], sem_w[b])
        pltpu.async_copy(qbuf.at[b, pl.ds(0, n)],
                         u2_hbm.at[pl.ds(off, n)], sem_w[b])

    def wait_wb(b, n=K):
        pltpu.make_async_copy(pbuf.at[b, pl.ds(0, n)],
                              u1_hbm.at[pl.ds(base, n)], sem_w[b]).wait()
        pltpu.make_async_copy(qbuf.at[b, pl.ds(0, n)],
                              u2_hbm.at[pl.ds(base, n)], sem_w[b]).wait()

    for b in range(GDIST):
        start_gather(b, b)

    def outer(g, carry):
        for b in range(NBUF):
            c = g * NBUF + b
            nxt = c + GDIST
            nb = (b + GDIST) % NBUF

            @pl.when(nxt < NCH)
            def _():
                @pl.when(nxt >= NBUF)
                def _():
                    wait_wb(nb)
                start_gather(nb, nxt)

            wait_gather(b)
            start_wb(b, c)
        return carry

    lax.fori_loop(0, NCH // NBUF, outer, 0)
    wait_wb(0)
    start_gather(0, NCH, TAIL)
    wait_gather(0, TAIL)
    start_wb(0, NCH, TAIL)
    wait_wb(1)
    wait_wb(2)
    wait_wb(0, TAIL)


@functools.partial(
    pl.kernel,
    out_type=jax.ShapeDtypeStruct((NC, N_PAD, D), jnp.float32),
    mesh=_sc_mesh,
    scratch_types=(
        [
            pltpu.VMEM((E_PER_W,), jnp.int32),
            pltpu.VMEM((SNBUF, K, D), jnp.float32),
            pltpu.VMEM_SHARED((N_PAD, D), jnp.float32),
        ]
        + [pltpu.SemaphoreType.DMA] * SNBUF
    ),
)
def _sc_scatter(msg_hbm, row_hbm, init_hbm, out_hbm, idx_v, mbuf, acc_sh, *sems):
    c = lax.axis_index("c")
    s = lax.axis_index("s")
    stripe = s * NODES_PER_TILE
    pltpu.sync_copy(init_hbm.at[c, pl.ds(stripe, NODES_PER_TILE)],
                    acc_sh.at[pl.ds(stripe, NODES_PER_TILE)])

    base = c * (EH // NC) + s * E_PER_W
    pltpu.sync_copy(row_hbm.at[pl.ds(base, E_PER_W)], idx_v)
    plsc.subcore_barrier()

    def start_load(b, chunk):
        pltpu.async_copy(msg_hbm.at[pl.ds(base + chunk * K, K)],
                         mbuf.at[b], sems[b])

    def wait_load(b):
        pltpu.make_async_copy(msg_hbm.at[pl.ds(base, K)], mbuf.at[b],
                              sems[b]).wait()

    def consume(b, chunk):
        wait_load(b)
        pltpu.sync_copy(mbuf.at[b],
                        acc_sh.at[idx_v.at[pl.ds(chunk * K, K)]], add=True)

    for b in range(SDIST):
        start_load(b, b)

    n_main = (NCH // SNBUF) * SNBUF

    def outer(g, carry):
        for b in range(SNBUF):
            chunk = g * SNBUF + b
            nxt = chunk + SDIST
            nb = (b + SDIST) % SNBUF

            @pl.when(nxt < NCH)
            def _():
                start_load(nb, nxt)

            consume(b, chunk)
        return carry

    lax.fori_loop(0, n_main // SNBUF, outer, 0)
    for chunk in range(n_main, NCH):
        consume(chunk % SNBUF, chunk)
    pltpu.sync_copy(msg_hbm.at[pl.ds(base + TAIL_OFF, TAIL)],
                    mbuf.at[1, pl.ds(0, TAIL)])
    pltpu.sync_copy(mbuf.at[1, pl.ds(0, TAIL)],
                    acc_sh.at[idx_v.at[pl.ds(TAIL_OFF, TAIL)]], add=True)
    plsc.subcore_barrier()
    pltpu.sync_copy(acc_sh.at[pl.ds(stripe, NODES_PER_TILE)],
                    out_hbm.at[c, pl.ds(stripe, NODES_PER_TILE)])


BM = 2000


def _pack_bf16(gate_part, core_part):
    gb = lax.bitcast_convert_type(
        gate_part.astype(jnp.bfloat16).astype(jnp.float32), jnp.uint32)
    cb = lax.bitcast_convert_type(
        core_part.astype(jnp.bfloat16).astype(jnp.float32), jnp.uint32)
    word = (gb >> 16) | (cb & jnp.uint32(0xFFFF0000))
    return lax.bitcast_convert_type(word, jnp.float32)


def _unpack_bf16(packed):
    w = lax.bitcast_convert_type(packed, jnp.uint32)
    gate_part = lax.bitcast_convert_type(w << 16, jnp.float32)
    core_part = lax.bitcast_convert_type(w & jnp.uint32(0xFFFF0000), jnp.float32)
    return gate_part, core_part


def _proj0_body(h_ref, wr_ref, wc_ref, p_ref, q_ref):
    h = h_ref[...]
    tp = jnp.dot(h, wr_ref[...], preferred_element_type=jnp.float32)
    tq = jnp.dot(h, wc_ref[...], preferred_element_type=jnp.float32)
    p_ref[...] = _pack_bf16(tp[:, :D], tp[:, D:])
    q_ref[...] = _pack_bf16(tq[:, :D], tq[:, D:])


def _proj_body(h_ref, a_ref, wr_ref, wc_ref, hn_ref, p_ref, q_ref):
    h = h_ref[...] + a_ref[0] + a_ref[1]
    hn_ref[...] = h
    tp = jnp.dot(h, wr_ref[...], preferred_element_type=jnp.float32)
    tq = jnp.dot(h, wc_ref[...], preferred_element_type=jnp.float32)
    p_ref[...] = _pack_bf16(tp[:, :D], tp[:, D:])
    q_ref[...] = _pack_bf16(tq[:, :D], tq[:, D:])


def _tc_proj0(h, wr, wc):
    return pl.pallas_call(
        _proj0_body,
        grid=(N_NODES // BM,),
        in_specs=[
            pl.BlockSpec((BM, D), lambda m: (m, 0)),
            pl.BlockSpec((D, 2 * D), lambda m: (0, 0)),
            pl.BlockSpec((D, 2 * D), lambda m: (0, 0)),
        ],
        out_specs=[
            pl.BlockSpec((BM, D), lambda m: (m, 0)),
            pl.BlockSpec((BM, D), lambda m: (m, 0)),
        ],
        out_shape=[
            jax.ShapeDtypeStruct((N_NODES, D), jnp.float32),
            jax.ShapeDtypeStruct((N_NODES, D), jnp.float32),
        ],
    )(h, wr, wc)


def _tc_proj(h, aggs, wr, wc):
    return pl.pallas_call(
        _proj_body,
        grid=(N_NODES // BM,),
        in_specs=[
            pl.BlockSpec((BM, D), lambda m: (m, 0)),
            pl.BlockSpec((NC, BM, D), lambda m: (0, m, 0)),
            pl.BlockSpec((D, 2 * D), lambda m: (0, 0)),
            pl.BlockSpec((D, 2 * D), lambda m: (0, 0)),
        ],
        out_specs=[
            pl.BlockSpec((BM, D), lambda m: (m, 0)),
            pl.BlockSpec((BM, D), lambda m: (m, 0)),
            pl.BlockSpec((BM, D), lambda m: (m, 0)),
        ],
        out_shape=[
            jax.ShapeDtypeStruct((N_NODES, D), jnp.float32),
            jax.ShapeDtypeStruct((N_NODES, D), jnp.float32),
            jax.ShapeDtypeStruct((N_NODES, D), jnp.float32),
        ],
    )(h, aggs, wr, wc)


BE = 4000


def _edge_body(u1_ref, u2_ref, e_ref, we_ref, b_ref, msg_ref):
    g1, c1 = _unpack_bf16(u1_ref[...])
    g2, c2 = _unpack_bf16(u2_ref[...])
    ep = (jnp.dot(e_ref[...], we_ref[...], preferred_element_type=jnp.float32)
          + b_ref[...])
    gate = jax.nn.sigmoid(g1 + g2 + ep[:, :D])
    core = jax.nn.softplus(c1 + c2 + ep[:, D:])
    msg_ref[...] = gate * core


def _tc_edge(u1, u2, e, we, bcat):
    return pl.pallas_call(
        _edge_body,
        grid=(EH // BE,),
        in_specs=[
            pl.BlockSpec((BE, D), lambda m: (m, 0)),
            pl.BlockSpec((BE, D), lambda m: (m, 0)),
            pl.BlockSpec((BE, D_EDGE), lambda m: (m, 0)),
            pl.BlockSpec((D_EDGE, 2 * D), lambda m: (0, 0)),
            pl.BlockSpec((1, 2 * D), lambda m: (0, 0)),
        ],
        out_specs=pl.BlockSpec((BE, D), lambda m: (m, 0)),
        out_shape=jax.ShapeDtypeStruct((EH, D), jnp.float32),
    )(u1, u2, e, we, bcat)


def _head_body(h_ref, a_ref, gid_ref, w1_ref, b1_ref, w2_ref, b2_ref,
               w3_ref, b3_ref, out_ref):
    h4 = h_ref[...] + a_ref[0, :N_NODES] + a_ref[1, :N_NODES]
    seg = lax.broadcasted_iota(jnp.int32, (N_GRAPHS, N_NODES), 0)
    m = (seg == gid_ref[...]).astype(jnp.float32)
    sums = jnp.dot(m, h4, preferred_element_type=jnp.float32)
    counts = jnp.sum(m, axis=1, keepdims=True)
    pooled = sums / jnp.maximum(counts, 1.0)
    o = jnp.maximum(jnp.dot(pooled, w1_ref[...],
                            preferred_element_type=jnp.float32) + b1_ref[...], 0.0)
    o = jnp.maximum(jnp.dot(o, w2_ref[...],
                            preferred_element_type=jnp.float32) + b2_ref[...], 0.0)
    out_ref[...] = jax.nn.sigmoid(
        jnp.dot(o, w3_ref[...], preferred_element_type=jnp.float32) + b3_ref[...])


def _tc_head(h, aggs, gid, w1, b1, w2, b2, w3, b3):
    return pl.pallas_call(
        _head_body,
        out_shape=jax.ShapeDtypeStruct((N_GRAPHS, 1), jnp.float32),
    )(h, aggs, gid, w1, b1, w2, b2, w3, b3)


def kernel(x, edge_index, e, i, Wf0, bf0, Ws0, bs0, Wf1, bf1, Ws1, bs1,
           Wf2, bf2, Ws2, bs2, W1, b1, W2, b2, W3, b3):
    row = edge_index[0]
    col = edge_index[1]
    row_h = (row[:EH], row[EH:])
    col_h = (col[:EH], col[EH:])
    e_h = (e[:EH], e[EH:])
    gid = i.reshape(1, N_NODES)
    zeros_parts = jnp.zeros((NC, N_PAD, D), jnp.float32)

    layers = []
    for Wf, bf, Ws, bs in ((Wf0, bf0, Ws0, bs0), (Wf1, bf1, Ws1, bs1),
                           (Wf2, bf2, Ws2, bs2)):
        wr = jnp.concatenate([Wf[:D], Ws[:D]], axis=1)
        wc = jnp.concatenate([Wf[D:2 * D], Ws[D:2 * D]], axis=1)
        we = jnp.concatenate([Wf[2 * D:], Ws[2 * D:]], axis=1)
        bcat = jnp.concatenate([bf, bs]).reshape(1, 2 * D)
        layers.append((wr, wc, we, bcat))

    h = x
    aggs = None
    for wr, wc, we, bcat in layers:
        if aggs is None:
            p, q = _tc_proj0(h, wr, wc)
        else:
            h, p, q = _tc_proj(h, aggs, wr, wc)
        u1a, u2a = _sc_gather(p, q, row_h[0], col_h[0])
        u1b, u2b = _sc_gather(p, q, row_h[1], col_h[1])
        msg_a = _tc_edge(u1a, u2a, e_h[0], we, bcat)
        msg_b = _tc_edge(u1b, u2b, e_h[1], we, bcat)
        part_a = _sc_scatter(msg_a, row_h[0], zeros_parts)
        aggs = _sc_scatter(msg_b, row_h[1], part_a)

    return _tc_head(h, aggs, gid, W1, b1.reshape(1, D), W2, b2.reshape(1, D),
                    W3, b3.reshape(1, 1))
